# baseline (speedup 1.0000x reference)
import functools

import jax
import jax.numpy as jnp
from jax import lax
from jax.experimental import pallas as pl
from jax.experimental.pallas import tpu as pltpu
from jax.experimental.pallas import tpu_sc as plsc

NC, NS, L = 2, 16, 16
NW = NC * NS
UNIT = 128
NBUF = 4
LEAD = 2


@functools.partial(jax.jit, static_argnums=(3, 4, 5))
def _sc_embed(x2d, table, pos, B, S, D):
    N = B * S
    PER_W = N // NW
    U = PER_W // UNIT
    G = U // NBUF

    mesh = plsc.VectorSubcoreMesh(core_axis_name="c", subcore_axis_name="s")

    @functools.partial(
        pl.kernel,
        mesh=mesh,
        out_type=jax.ShapeDtypeStruct((N, D), jnp.float32),
        scratch_types=[
            pltpu.VMEM((U, UNIT), jnp.int32),
            pltpu.VMEM((2 * S, D), jnp.float32),
            pltpu.VMEM((NBUF, UNIT, D), jnp.float32),
            pltpu.SemaphoreType.DMA((NBUF,)),
            pltpu.SemaphoreType.DMA((NBUF,)),
        ],
    )
    def body(x_hbm, tab_hbm, pos_hbm, out_hbm, idx_v, pos_v, buf_v, gsem, ssem):
        wid = lax.axis_index("s") * NC + lax.axis_index("c")
        pltpu.sync_copy(x_hbm.at[wid], idx_v)
        pltpu.sync_copy(pos_hbm.at[pl.ds(0, S)], pos_v.at[pl.ds(0, S)])
        pltpu.sync_copy(pos_hbm.at[pl.ds(0, S)], pos_v.at[pl.ds(S, S)])
        out_base = wid * PER_W

        def gather(u, k):
            return pltpu.make_async_copy(
                tab_hbm.at[idx_v.at[u]], buf_v.at[k], gsem.at[k])

        def scatter(u, k):
            return pltpu.make_async_copy(
                buf_v.at[k],
                out_hbm.at[pl.ds(out_base + u * UNIT, UNIT)], ssem.at[k])

        def unit(u, k, issue_next, wait_reuse):
            gather(u, k).wait()
            if issue_next:
                kv = (k + LEAD) % NBUF
                if wait_reuse:
                    scatter(u - LEAD, kv).wait()
                gather(u + LEAD, kv).start()
            off = lax.rem(u * UNIT, S)

            def add_row(r, carry):
                for j in range(D // L):
                    sl = pl.ds(j * L, L)
                    plsc.addupdate(buf_v.at[k, r, sl], pos_v[off + r, sl])
                return carry

            lax.fori_loop(0, UNIT, add_row, 0)
            scatter(u, k).start()

        for b in range(LEAD):
            gather(b, b).start()
        for k in range(NBUF):
            unit(k, k, True, k >= LEAD)

        def group(gg, carry):
            for k in range(NBUF):
                unit(gg * NBUF + k, k, True, True)
            return carry

        lax.fori_loop(1, G, group, 0)

        for k in range(U - G * NBUF):
            u = G * NBUF + k
            unit(u, k, u + LEAD < U, True)
        for k in range(NBUF):
            scatter(U - NBUF + k, (U - NBUF + k) % NBUF).wait()

    return body(x2d, table, pos)


def kernel(x, embed_table, pos_table):
    B, S = x.shape
    D = embed_table.shape[1]
    x2d = x.reshape(NW, B * S // (NW * UNIT), UNIT)
    out = _sc_embed(x2d, embed_table, pos_table, B, S, D)
    return out.reshape(B, S, D)


# speedup vs baseline: 1.7846x; 1.7846x over previous
import functools

import jax
import jax.numpy as jnp
from jax import lax
from jax.experimental import pallas as pl
from jax.experimental.pallas import tpu as pltpu
from jax.experimental.pallas import tpu_sc as plsc

NC, NS, L = 2, 16, 16
NW = NC * NS
WIN = 8
SEQG = 16
UNIT = WIN * SEQG
NBUF = 4
LEAD = 2


@functools.partial(jax.jit, static_argnums=(3, 4, 5))
def _sc_embed(xr, table, pos, B, S, D):
    N = B * S
    PER_W = N // NW
    U = PER_W // UNIT
    G = U // NBUF
    NWIN = S // WIN
    SEQ_PER_W = B // NW

    mesh = plsc.VectorSubcoreMesh(core_axis_name="c", subcore_axis_name="s")

    @functools.partial(
        pl.kernel,
        mesh=mesh,
        out_type=jax.ShapeDtypeStruct((B, S, D), jnp.float32),
        scratch_types=[
            pltpu.VMEM((U, UNIT), jnp.int32),
            pltpu.VMEM((S, D), jnp.float32),
            pltpu.VMEM((NBUF, UNIT, D), jnp.float32),
            pltpu.SemaphoreType.DMA((NBUF,)),
            pltpu.SemaphoreType.DMA((NBUF,)),
        ],
    )
    def body(x_hbm, tab_hbm, pos_hbm, out_hbm, idx_v, pos_v, buf_v, gsem, ssem):
        wid = lax.axis_index("s") * NC + lax.axis_index("c")
        pltpu.sync_copy(x_hbm.at[wid], idx_v)
        pltpu.sync_copy(pos_hbm.at[pl.ds(0, S)], pos_v)
        seq_base = wid * SEQ_PER_W

        def gather(u, k):
            return pltpu.make_async_copy(
                tab_hbm.at[idx_v.at[u]], buf_v.at[k], gsem.at[k])

        def scatter_start(u, k):
            sg = lax.div(u, NWIN)
            p0 = lax.rem(u, NWIN) * WIN
            seq0 = seq_base + sg * SEQG
            for q in range(SEQG):
                pltpu.make_async_copy(
                    buf_v.at[k, pl.ds(q * WIN, WIN)],
                    out_hbm.at[seq0 + q, pl.ds(p0, WIN)],
                    ssem.at[k]).start()

        def scatter_wait(k):
            pltpu.make_async_copy(
                tab_hbm.at[pl.ds(0, UNIT)], buf_v.at[k], ssem.at[k]).wait()

        def unit(u, k, issue_next, wait_reuse):
            gather(u, k).wait()
            if issue_next:
                kv = (k + LEAD) % NBUF
                if wait_reuse:
                    scatter_wait(kv)
                gather(u + LEAD, kv).start()
            p0 = lax.rem(u, NWIN) * WIN

            for rr in range(WIN):
                prow = [pos_v[p0 + rr, pl.ds(j * L, L)] for j in range(D // L)]

                def add_q(q, carry, rr=rr, prow=prow):
                    for j in range(D // L):
                        plsc.addupdate(
                            buf_v.at[k, q * WIN + rr, pl.ds(j * L, L)],
                            prow[j])
                    return carry

                lax.fori_loop(0, SEQG, add_q, 0)
            scatter_start(u, k)

        for b in range(LEAD):
            gather(b, b).start()
        for k in range(NBUF):
            unit(k, k, True, k >= LEAD)

        def group(gg, carry):
            for k in range(NBUF):
                unit(gg * NBUF + k, k, True, True)
            return carry

        lax.fori_loop(1, G, group, 0)

        for k in range(U - G * NBUF):
            u = G * NBUF + k
            unit(u, k, u + LEAD < U, True)
        for k in range(NBUF):
            scatter_wait(k)

    return body(xr, table, pos)


def kernel(x, embed_table, pos_table):
    B, S = x.shape
    D = embed_table.shape[1]
    xr = x.reshape(NW, B // (NW * SEQG), SEQG, S // WIN, WIN)
    xr = xr.transpose(0, 1, 3, 2, 4).reshape(NW, B * S // (NW * UNIT), UNIT)
    return _sc_embed(xr, embed_table, pos_table, B, S, D)


# speedup vs baseline: 1.7847x; 1.0001x over previous
import functools

import jax
import jax.numpy as jnp
from jax import lax
from jax.experimental import pallas as pl
from jax.experimental.pallas import tpu as pltpu
from jax.experimental.pallas import tpu_sc as plsc

NC, NS, L = 2, 16, 16
NW = NC * NS
WIN = 8
SEQG = 16
UNIT = WIN * SEQG
NBUF = 4
LEAD = 2


@functools.partial(jax.jit, static_argnums=(3, 4, 5))
def _sc_embed(xr, table, pos, B, S, D):
    N = B * S
    PER_W = N // NW
    U = PER_W // UNIT
    G = U // NBUF
    NWIN = S // WIN
    SEQ_PER_W = B // NW

    mesh = plsc.VectorSubcoreMesh(core_axis_name="c", subcore_axis_name="s")

    @functools.partial(
        pl.kernel,
        mesh=mesh,
        out_type=jax.ShapeDtypeStruct((B, S, D), jnp.float32),
        scratch_types=[
            pltpu.VMEM((U, UNIT), jnp.int32),
            pltpu.VMEM((S, D), jnp.float32),
            pltpu.VMEM((NBUF, UNIT, D), jnp.float32),
            pltpu.SemaphoreType.DMA((NBUF,)),
            pltpu.SemaphoreType.DMA((NBUF,)),
        ],
    )
    def body(x_hbm, tab_hbm, pos_hbm, out_hbm, idx_v, pos_v, buf_v, gsem, ssem):
        wid = lax.axis_index("s") * NC + lax.axis_index("c")
        pltpu.sync_copy(x_hbm.at[wid], idx_v)
        pltpu.sync_copy(pos_hbm.at[pl.ds(0, S)], pos_v)
        seq_base = wid * SEQ_PER_W

        def gather(u, k):
            return pltpu.make_async_copy(
                tab_hbm.at[idx_v.at[u]], buf_v.at[k], gsem.at[k])

        def scatter_start(u, k):
            sg = lax.div(u, NWIN)
            p0 = lax.rem(u, NWIN) * WIN
            seq0 = seq_base + sg * SEQG
            pltpu.make_async_copy(
                buf_v.at[k].reshape(SEQG, WIN, D),
                out_hbm.at[pl.ds(seq0, SEQG), pl.ds(p0, WIN)],
                ssem.at[k]).start()

        def scatter_wait(k):
            pltpu.make_async_copy(
                tab_hbm.at[pl.ds(0, UNIT)], buf_v.at[k], ssem.at[k]).wait()

        def unit(u, k, issue_next, wait_reuse):
            gather(u, k).wait()
            if issue_next:
                kv = (k + LEAD) % NBUF
                if wait_reuse:
                    scatter_wait(kv)
                gather(u + LEAD, kv).start()
            p0 = lax.rem(u, NWIN) * WIN

            for rr in range(WIN):
                prow = [pos_v[p0 + rr, pl.ds(j * L, L)] for j in range(D // L)]

                def add_q(q, carry, rr=rr, prow=prow):
                    for j in range(D // L):
                        plsc.addupdate(
                            buf_v.at[k, q * WIN + rr, pl.ds(j * L, L)],
                            prow[j])
                    return carry

                lax.fori_loop(0, SEQG, add_q, 0)
            scatter_start(u, k)

        for b in range(LEAD):
            gather(b, b).start()
        for k in range(NBUF):
            unit(k, k, True, k >= LEAD)

        def group(gg, carry):
            for k in range(NBUF):
                unit(gg * NBUF + k, k, True, True)
            return carry

        lax.fori_loop(1, G, group, 0)

        for k in range(U - G * NBUF):
            u = G * NBUF + k
            unit(u, k, u + LEAD < U, True)
        for k in range(NBUF):
            scatter_wait(k)

    return body(xr, table, pos)


def kernel(x, embed_table, pos_table):
    B, S = x.shape
    D = embed_table.shape[1]
    xr = x.reshape(NW, B // (NW * SEQG), SEQG, S // WIN, WIN)
    xr = xr.transpose(0, 1, 3, 2, 4).reshape(NW, B * S // (NW * UNIT), UNIT)
    return _sc_embed(xr, embed_table, pos_table, B, S, D)
